# baseline (speedup 1.0000x reference)
import functools

import jax
import jax.numpy as jnp
from jax.experimental import pallas as pl
from jax.experimental.pallas import tpu as pltpu

_S = 64
_ROWS_PER_B = 32
_NGRP = 16
_NBLK = 32


def _pool_mm_kernel(x_ref, wl_ref, wc_ref, bl_ref, bc_ref,
                    ol_ref, oc_ref, *, g):
    i = pl.program_id(0)
    blk = x_ref[...]
    blk = blk.reshape(g, _ROWS_PER_B, 2 * _NGRP, 128)
    s1 = jnp.sum(blk, axis=1)
    s2 = s1.reshape(g, _NGRP, 2, 128).sum(axis=2)
    sb = (s2 * (1.0 / _S)).astype(jnp.bfloat16)
    wl = wl_ref[...].astype(jnp.bfloat16)
    wc = wc_ref[...].astype(jnp.bfloat16)
    accl = jnp.broadcast_to(bl_ref[...], (2, g))
    accc = jnp.broadcast_to(bc_ref[...], (3, g))
    dn = (((1,), (1,)), ((), ()))
    for grp in range(_NGRP):
        sl = slice(grp * 128, (grp + 1) * 128)
        accl = accl + jax.lax.dot_general(
            wl[:, sl], sb[:, grp, :], dn,
            preferred_element_type=jnp.float32)
        accc = accc + jax.lax.dot_general(
            wc[:, sl], sb[:, grp, :], dn,
            preferred_element_type=jnp.float32)
    for k in range(_NBLK):
        @pl.when(i == k)
        def _store():
            ol_ref[:, k * g:(k + 1) * g] = accl
            oc_ref[:, k * g:(k + 1) * g] = accc


def kernel(x, start_boundaries, W_loc, b_loc, W_conf, b_conf):
    B, C = x.shape[0], x.shape[1]
    xt = (x.transpose(0, 2, 3, 4, 1)
          .reshape(B * _ROWS_PER_B, 2, _NGRP, 128)
          .transpose(0, 2, 1, 3)
          .reshape(B * _ROWS_PER_B, 2 * _NGRP, 128))

    g = B // _NBLK
    rb = g * _ROWS_PER_B
    out2, out3 = pl.pallas_call(
        functools.partial(_pool_mm_kernel, g=g),
        grid=(_NBLK,),
        in_specs=[
            pl.BlockSpec((rb, 2 * _NGRP, 128), lambda i: (i, 0, 0)),
            pl.BlockSpec((2, C), lambda i: (0, 0)),
            pl.BlockSpec((3, C), lambda i: (0, 0)),
            pl.BlockSpec((2, 1), lambda i: (0, 0)),
            pl.BlockSpec((3, 1), lambda i: (0, 0)),
        ],
        out_specs=[
            pl.BlockSpec((2, B), lambda i: (0, 0)),
            pl.BlockSpec((3, B), lambda i: (0, 0)),
        ],
        out_shape=[
            jax.ShapeDtypeStruct((2, B), jnp.float32),
            jax.ShapeDtypeStruct((3, B), jnp.float32),
        ],
        compiler_params=pltpu.CompilerParams(
            dimension_semantics=("arbitrary",)),
    )(xt, W_loc, W_conf, b_loc.reshape(2, 1), b_conf.reshape(3, 1))

    return out2.T, out3.T


# speedup vs baseline: 1.0566x; 1.0566x over previous
import functools

import jax
import jax.numpy as jnp
from jax.experimental import pallas as pl
from jax.experimental.pallas import tpu as pltpu

_S = 64
_ROWS_PER_B = 32
_NGRP = 16
_NBLK = 16


def _pool_mm_kernel(x_ref, wl_ref, wc_ref, bl_ref, bc_ref,
                    ol_ref, oc_ref, *, g):
    i = pl.program_id(0)
    blk = x_ref[...]
    blk = blk.reshape(g, _ROWS_PER_B, 2 * _NGRP, 128)
    s1 = jnp.sum(blk, axis=1)
    s2 = s1.reshape(g, _NGRP, 2, 128).sum(axis=2)
    sb = (s2 * (1.0 / _S)).astype(jnp.bfloat16)
    wl = wl_ref[...].astype(jnp.bfloat16)
    wc = wc_ref[...].astype(jnp.bfloat16)
    accl = jnp.broadcast_to(bl_ref[...], (2, g))
    accc = jnp.broadcast_to(bc_ref[...], (3, g))
    dn = (((1,), (1,)), ((), ()))
    for grp in range(_NGRP):
        sl = slice(grp * 128, (grp + 1) * 128)
        accl = accl + jax.lax.dot_general(
            wl[:, sl], sb[:, grp, :], dn,
            preferred_element_type=jnp.float32)
        accc = accc + jax.lax.dot_general(
            wc[:, sl], sb[:, grp, :], dn,
            preferred_element_type=jnp.float32)
    for k in range(_NBLK):
        @pl.when(i == k)
        def _store():
            ol_ref[:, k * g:(k + 1) * g] = accl
            oc_ref[:, k * g:(k + 1) * g] = accc


def kernel(x, start_boundaries, W_loc, b_loc, W_conf, b_conf):
    B, C = x.shape[0], x.shape[1]
    xt = (x.transpose(0, 2, 3, 4, 1)
          .reshape(B * _ROWS_PER_B, 2, _NGRP, 128)
          .transpose(0, 2, 1, 3)
          .reshape(B * _ROWS_PER_B, 2 * _NGRP, 128))

    g = B // _NBLK
    rb = g * _ROWS_PER_B
    out2, out3 = pl.pallas_call(
        functools.partial(_pool_mm_kernel, g=g),
        grid=(_NBLK,),
        in_specs=[
            pl.BlockSpec((rb, 2 * _NGRP, 128), lambda i: (i, 0, 0)),
            pl.BlockSpec((2, C), lambda i: (0, 0)),
            pl.BlockSpec((3, C), lambda i: (0, 0)),
            pl.BlockSpec((2, 1), lambda i: (0, 0)),
            pl.BlockSpec((3, 1), lambda i: (0, 0)),
        ],
        out_specs=[
            pl.BlockSpec((2, B), lambda i: (0, 0)),
            pl.BlockSpec((3, B), lambda i: (0, 0)),
        ],
        out_shape=[
            jax.ShapeDtypeStruct((2, B), jnp.float32),
            jax.ShapeDtypeStruct((3, B), jnp.float32),
        ],
        compiler_params=pltpu.CompilerParams(
            dimension_semantics=("arbitrary",)),
    )(xt, W_loc, W_conf, b_loc.reshape(2, 1), b_conf.reshape(3, 1))

    return out2.T, out3.T
